# baseline (speedup 1.0000x reference)
import functools

import jax
import jax.numpy as jnp
from jax import lax
from jax.experimental import pallas as pl
from jax.experimental.pallas import tpu as pltpu
from jax.experimental.pallas import tpu_sc as plsc

N = 10000
F = 128
E = 320000
NC = 2
NS = 16
NW = NC * NS
CH = 50
PER_TILE = E // NW
STEPS = PER_TILE // CH
NBUF = 4
ROWS_PER_SUB = N // NS


def _agg_body(h_hbm, src_hbm, dst_hbm, out_hbm,
              src_v, dst_v, rows, acc, gsems, ssems):
  c = lax.axis_index("c")
  s = lax.axis_index("s")
  widx = c * NS + s

  row0 = s * ROWS_PER_SUB
  for t in range(13):
    r = row0 + t * CH
    n = CH if t < 12 else ROWS_PER_SUB - 12 * CH
    pltpu.sync_copy(h_hbm.at[pl.ds(r, n)], rows.at[0, pl.ds(0, n)])
    pltpu.sync_copy(rows.at[0, pl.ds(0, n)], acc.at[pl.ds(r, n)])
  plsc.subcore_barrier()

  pltpu.sync_copy(src_hbm.at[pl.ds(widx * STEPS, STEPS)], src_v)
  pltpu.sync_copy(dst_hbm.at[pl.ds(widx * STEPS, STEPS)], dst_v)

  def gather(m, k):
    pltpu.async_copy(h_hbm.at[src_v.at[m]], rows.at[k], gsems.at[k])

  def scatter_start(m, k):
    pltpu.async_copy(rows.at[k], acc.at[dst_v.at[m]], ssems.at[k], add=True)

  def gwait(m, k):
    pltpu.make_async_copy(h_hbm.at[src_v.at[m]], rows.at[k], gsems.at[k]).wait()

  def swait(m, k):
    pltpu.make_async_copy(rows.at[k], acc.at[dst_v.at[m]], ssems.at[k]).wait()

  gather(0, 0)
  gather(1, 1)

  def body(i, carry):
    m0 = i * NBUF
    for k in range(NBUF):
      m = m0 + k
      @pl.when(m - 2 >= 0)
      def _():
        swait(m - 2, (k + 2) % NBUF)
      @pl.when(m + 2 < STEPS)
      def _():
        gather(m + 2, (k + 2) % NBUF)
      gwait(m, k)
      scatter_start(m, k)
    return carry

  lax.fori_loop(0, STEPS // NBUF, body, 0)
  swait(STEPS - 2, (STEPS - 2) % NBUF)
  swait(STEPS - 1, (STEPS - 1) % NBUF)

  plsc.subcore_barrier()
  for t in range(13):
    r = row0 + t * CH
    n = CH if t < 12 else ROWS_PER_SUB - 12 * CH
    pltpu.sync_copy(acc.at[pl.ds(r, n)], rows.at[0, pl.ds(0, n)])
    pltpu.sync_copy(rows.at[0, pl.ds(0, n)], out_hbm.at[c, pl.ds(r, n)])


_agg = pl.kernel(
    _agg_body,
    out_type=jax.ShapeDtypeStruct((NC, N, F), jnp.float32),
    mesh=plsc.VectorSubcoreMesh(core_axis_name="c", subcore_axis_name="s"),
    scratch_types=[
        pltpu.VMEM((STEPS, CH), jnp.int32),
        pltpu.VMEM((STEPS, CH), jnp.int32),
        pltpu.VMEM((NBUF, CH, F), jnp.float32),
        pltpu.VMEM_SHARED((N, F), jnp.float32),
        pltpu.SemaphoreType.DMA((NBUF,)),
        pltpu.SemaphoreType.DMA((NBUF,)),
    ],
    compiler_params=pltpu.CompilerParams(use_tc_tiling_on_sc=False),
)


def _mlp_bn_body(pA, pB, h, Wa, ba, Wb, bb, g, be, out):
  z = pA[...] + pB[...] - h[...]
  u = jnp.maximum(jnp.dot(z, Wa[...], preferred_element_type=jnp.float32)
                  + ba[...], 0.0)
  v = jnp.dot(u, Wb[...], preferred_element_type=jnp.float32) + bb[...]
  v = jnp.maximum(v, 0.0)
  m = jnp.mean(v, axis=0, keepdims=True)
  var = jnp.mean((v - m) * (v - m), axis=0, keepdims=True)
  out[...] = (v - m) * lax.rsqrt(var + 1e-5) * g[...] + be[...]


def _mlp_bn(parts, h, Wa, ba, Wb, bb, g, be, dout):
  return pl.pallas_call(
      _mlp_bn_body,
      out_shape=jax.ShapeDtypeStruct((N, dout), jnp.float32),
  )(parts[0], parts[1], h, Wa, ba, Wb, bb, g, be)


def kernel(x, edge_index, W1a, b1a, W1b, b1b, g1, be1,
           W2a, b2a, W2b, b2b, g2, be2):
  src2 = edge_index[0].reshape(NW * STEPS, CH)
  dst2 = edge_index[1].reshape(NW * STEPS, CH)
  parts1 = _agg(x, src2, dst2)
  h1 = _mlp_bn(parts1, x, W1a, b1a, W1b, b1b, g1, be1, F)
  parts2 = _agg(h1, src2, dst2)
  h2 = _mlp_bn(parts2, h1, W2a, b2a, W2b, b2b, g2, be2, 2)
  return h2


# speedup vs baseline: 1.0722x; 1.0722x over previous
import functools

import jax
import jax.numpy as jnp
from jax import lax
from jax.experimental import pallas as pl
from jax.experimental.pallas import tpu as pltpu
from jax.experimental.pallas import tpu_sc as plsc

N = 10000
F = 128
E = 320000
NC = 2
NS = 16
FH = F // NC
CH = 40
PER_TILE = E // NS
STEPS = PER_TILE // CH
NBUF = 4
ROWS_PER_SUB = N // NS


def _agg_body(h_hbm, src_hbm, dst_hbm, out_hbm,
              src_v, dst_v, rows, acc, tab, gsems, ssems):
  c = lax.axis_index("c")
  s = lax.axis_index("s")
  col0 = c * FH

  row0 = s * ROWS_PER_SUB
  for t in range(16):
    r = row0 + t * CH
    n = CH if t < 15 else ROWS_PER_SUB - 15 * CH
    pltpu.sync_copy(h_hbm.at[pl.ds(r, n), pl.ds(col0, FH)],
                    rows.at[0, pl.ds(0, n)])
    pltpu.sync_copy(rows.at[0, pl.ds(0, n)], tab.at[pl.ds(r, n)])
    pltpu.sync_copy(rows.at[0, pl.ds(0, n)], acc.at[pl.ds(r, n)])
  plsc.subcore_barrier()

  pltpu.sync_copy(src_hbm.at[pl.ds(s * STEPS, STEPS)], src_v)
  pltpu.sync_copy(dst_hbm.at[pl.ds(s * STEPS, STEPS)], dst_v)

  def gather(m, k):
    pltpu.async_copy(tab.at[src_v.at[m]], rows.at[k], gsems.at[k])

  def scatter_start(m, k):
    pltpu.async_copy(rows.at[k], acc.at[dst_v.at[m]], ssems.at[k], add=True)

  def gwait(m, k):
    pltpu.make_async_copy(tab.at[src_v.at[m]], rows.at[k], gsems.at[k]).wait()

  def swait(m, k):
    pltpu.make_async_copy(rows.at[k], acc.at[dst_v.at[m]], ssems.at[k]).wait()

  gather(0, 0)
  gather(1, 1)

  def body(i, carry):
    m0 = i * NBUF
    for k in range(NBUF):
      m = m0 + k
      @pl.when(m - 2 >= 0)
      def _():
        swait(m - 2, (k + 2) % NBUF)
      @pl.when(m + 2 < STEPS)
      def _():
        gather(m + 2, (k + 2) % NBUF)
      gwait(m, k)
      scatter_start(m, k)
    return carry

  lax.fori_loop(0, STEPS // NBUF, body, 0)
  swait(STEPS - 2, (STEPS - 2) % NBUF)
  swait(STEPS - 1, (STEPS - 1) % NBUF)

  plsc.subcore_barrier()
  for t in range(16):
    r = row0 + t * CH
    n = CH if t < 15 else ROWS_PER_SUB - 15 * CH
    pltpu.sync_copy(acc.at[pl.ds(r, n)], rows.at[0, pl.ds(0, n)])
    pltpu.sync_copy(rows.at[0, pl.ds(0, n)],
                    out_hbm.at[pl.ds(r, n), pl.ds(col0, FH)])


_agg = pl.kernel(
    _agg_body,
    out_type=jax.ShapeDtypeStruct((N, F), jnp.float32),
    mesh=plsc.VectorSubcoreMesh(core_axis_name="c", subcore_axis_name="s"),
    scratch_types=[
        pltpu.VMEM((STEPS, CH), jnp.int32),
        pltpu.VMEM((STEPS, CH), jnp.int32),
        pltpu.VMEM((NBUF, CH, FH), jnp.float32),
        pltpu.VMEM_SHARED((N, FH), jnp.float32),
        pltpu.VMEM_SHARED((N, FH), jnp.float32),
        pltpu.SemaphoreType.DMA((NBUF,)),
        pltpu.SemaphoreType.DMA((NBUF,)),
    ],
    compiler_params=pltpu.CompilerParams(use_tc_tiling_on_sc=False),
)


def _mlp_bn_body(z, Wa, ba, Wb, bb, g, be, out):
  u = jnp.maximum(jnp.dot(z[...], Wa[...], preferred_element_type=jnp.float32)
                  + ba[...], 0.0)
  v = jnp.dot(u, Wb[...], preferred_element_type=jnp.float32) + bb[...]
  v = jnp.maximum(v, 0.0)
  m = jnp.mean(v, axis=0, keepdims=True)
  var = jnp.mean((v - m) * (v - m), axis=0, keepdims=True)
  out[...] = (v - m) * lax.rsqrt(var + 1e-5) * g[...] + be[...]


def _mlp_bn(z, Wa, ba, Wb, bb, g, be, dout):
  return pl.pallas_call(
      _mlp_bn_body,
      out_shape=jax.ShapeDtypeStruct((N, dout), jnp.float32),
  )(z, Wa, ba, Wb, bb, g, be)


def kernel(x, edge_index, W1a, b1a, W1b, b1b, g1, be1,
           W2a, b2a, W2b, b2b, g2, be2):
  src2 = edge_index[0].reshape(NS * STEPS, CH)
  dst2 = edge_index[1].reshape(NS * STEPS, CH)
  z1 = _agg(x, src2, dst2)
  h1 = _mlp_bn(z1, W1a, b1a, W1b, b1b, g1, be1, F)
  z2 = _agg(h1, src2, dst2)
  h2 = _mlp_bn(z2, W2a, b2a, W2b, b2b, g2, be2, 2)
  return h2
